# Initial kernel scaffold; baseline (speedup 1.0000x reference)
import jax
import jax.numpy as jnp
from jax.experimental import pallas as pl


def kernel(batch_video, batch_audio, batch_text, batch_target, batch_group):
    raise NotImplementedError("write your pallas kernel here")


# speedup vs baseline: 1.3155x; 1.3155x over previous
import dataclasses
import functools

import jax
import jax.numpy as jnp
from jax import lax
from jax.experimental import pallas as pl
from jax.experimental.pallas import tpu as pltpu
from jax.experimental.pallas import tpu_sc as plsc

BATCH = 1024
NUM_GROUP = 2
TAU = 0.2
VD = 16 * 768
AD = 128
TD = 768

NC = 2
NS = 16
L = 16
NW = NC * NS
BPW = BATCH // NW
VCHUNK = 8
TCHUNK = 16


def _argmax_body(g_ref, logit_ref, out_ref):
    v = g_ref[...] + logit_ref[...]
    m = jnp.max(v, axis=1, keepdims=True)
    cols = lax.broadcasted_iota(jnp.int32, v.shape, 1)
    masked = jnp.where(v == m, cols, BATCH)
    out_ref[...] = jnp.min(masked, axis=1, keepdims=True)


def _tc_argmax(gumbel_field, logit_row):
    return pl.pallas_call(
        _argmax_body,
        out_shape=jax.ShapeDtypeStruct((BATCH, 1), jnp.int32),
    )(gumbel_field, logit_row)


def _sc_gather_body(video_hbm, audio_hbm, text_hbm, tgt_hbm, grp_hbm, idx_hbm,
                    shuf_hbm, v_out, a_out, t_out, tg_out, gr_out,
                    idx_v, tgt_v, grp_v, shuf_v, fused_v, tgo_v, gro_v,
                    vbuf, abuf, tbuf):
    wid = lax.axis_index("s") * NC + lax.axis_index("c")
    base = wid * BPW

    pltpu.sync_copy(idx_hbm, idx_v)
    pltpu.sync_copy(tgt_hbm, tgt_v)
    pltpu.sync_copy(grp_hbm, grp_v)
    pltpu.sync_copy(shuf_hbm.at[pl.ds(base, BPW)], shuf_v)

    @pl.loop(0, BPW, step=L)
    def _(k):
        sh = shuf_v[pl.ds(k, L)]
        f = plsc.load_gather(idx_v, [sh])
        fused_v[pl.ds(k, L)] = f
        tgo_v[pl.ds(k, L)] = plsc.load_gather(tgt_v, [f])
        gro_v[pl.ds(k, L)] = plsc.load_gather(grp_v, [f])

    pltpu.sync_copy(tgo_v, tg_out.at[pl.ds(base, BPW)])
    pltpu.sync_copy(gro_v, gr_out.at[pl.ds(base, BPW)])

    @pl.loop(0, BPW, step=VCHUNK)
    def _(c):
        pltpu.sync_copy(video_hbm.at[fused_v.at[pl.ds(c, VCHUNK)]], vbuf)
        pltpu.sync_copy(vbuf, v_out.at[pl.ds(base + c, VCHUNK)])

    pltpu.sync_copy(audio_hbm.at[fused_v], abuf)
    pltpu.sync_copy(abuf, a_out.at[pl.ds(base, BPW)])

    @pl.loop(0, BPW, step=TCHUNK)
    def _(c):
        pltpu.sync_copy(text_hbm.at[fused_v.at[pl.ds(c, TCHUNK)]], tbuf)
        pltpu.sync_copy(tbuf, t_out.at[pl.ds(base + c, TCHUNK)])


@functools.cache
def _sc_gather_kernel():
    mesh = plsc.VectorSubcoreMesh(core_axis_name="c", subcore_axis_name="s")
    cp = pltpu.CompilerParams()
    if "needs_layout_passes" in pltpu.CompilerParams.__dataclass_fields__:
        cp = dataclasses.replace(cp, needs_layout_passes=False)
    return pl.kernel(
        _sc_gather_body,
        compiler_params=cp,
        out_type=[
        jax.ShapeDtypeStruct((BATCH, VD), jnp.float32),
        jax.ShapeDtypeStruct((BATCH, AD), jnp.float32),
        jax.ShapeDtypeStruct((BATCH, TD), jnp.float32),
        jax.ShapeDtypeStruct((BATCH,), jnp.int32),
        jax.ShapeDtypeStruct((BATCH,), jnp.int32),
    ],
    mesh=mesh,
    scratch_types=[
        pltpu.VMEM((BATCH,), jnp.int32),
        pltpu.VMEM((BATCH,), jnp.int32),
        pltpu.VMEM((BATCH,), jnp.int32),
        pltpu.VMEM((BPW,), jnp.int32),
        pltpu.VMEM((BPW,), jnp.int32),
        pltpu.VMEM((BPW,), jnp.int32),
        pltpu.VMEM((BPW,), jnp.int32),
        pltpu.VMEM((VCHUNK, VD), jnp.float32),
        pltpu.VMEM((BPW, AD), jnp.float32),
        pltpu.VMEM((TCHUNK, TD), jnp.float32),
    ],
)


def kernel(batch_video, batch_audio, batch_text, batch_target, batch_group):
    n_c = batch_group.shape[0]
    counts = jnp.bincount(batch_group, length=NUM_GROUP)
    weights = (counts.astype(jnp.float32) / n_c) ** TAU
    weights = weights / weights.sum()
    logw = jnp.log(weights)
    logit_row = logw[batch_group][None, :]

    key = jax.random.key(42)
    k_mult, k_perm = jax.random.split(key)
    gumbel_field = jax.random.gumbel(k_mult, (BATCH, BATCH), jnp.float32)
    shuffle_idx = jax.random.permutation(k_perm, BATCH).astype(jnp.int32)

    indices = _tc_argmax(gumbel_field, logit_row).reshape(BATCH)

    video2 = batch_video.reshape(BATCH, VD)
    v, a, t, tg, gr = _sc_gather_kernel()(video2, batch_audio, batch_text,
                                          batch_target, batch_group, indices,
                                          shuffle_idx)
    return (v.reshape(BATCH, 16, 768), a, t, tg, gr)


# speedup vs baseline: 1.8328x; 1.3932x over previous
import dataclasses
import functools

import jax
import jax.numpy as jnp
from jax import lax
from jax.experimental import pallas as pl
from jax.experimental.pallas import tpu as pltpu
from jax.experimental.pallas import tpu_sc as plsc

BATCH = 1024
NUM_GROUP = 2
TAU = 0.2
VD = 16 * 768
AD = 128
TD = 768

NC = 2
NS = 16
L = 16
NW = NC * NS
BPW = BATCH // NW
VCHUNK = 8
TCHUNK = 16


def _argmax_body(g_ref, logit_ref, out_ref):
    v = g_ref[...] + logit_ref[...]
    m = jnp.max(v, axis=1, keepdims=True)
    cols = lax.broadcasted_iota(jnp.int32, v.shape, 1)
    masked = jnp.where(v == m, cols, BATCH)
    out_ref[...] = jnp.min(masked, axis=1, keepdims=True)


def _tc_argmax(gumbel_field, logit_row):
    return pl.pallas_call(
        _argmax_body,
        out_shape=jax.ShapeDtypeStruct((BATCH, 1), jnp.int32),
    )(gumbel_field, logit_row)


def _sc_gather_body(video_hbm, audio_hbm, text_hbm, tgt_hbm, grp_hbm, idx_hbm,
                    shuf_hbm, v_out, a_out, t_out, tg_out, gr_out,
                    idx_v, tgt_v, grp_v, shuf_v, fused_v, tgo_v, gro_v,
                    vbuf, abuf, tbuf):
    wid = lax.axis_index("s") * NC + lax.axis_index("c")
    base = wid * BPW

    pltpu.sync_copy(idx_hbm, idx_v)
    pltpu.sync_copy(tgt_hbm, tgt_v)
    pltpu.sync_copy(grp_hbm, grp_v)
    pltpu.sync_copy(shuf_hbm.at[pl.ds(base, BPW)], shuf_v)

    @pl.loop(0, BPW, step=L)
    def _(k):
        sh = shuf_v[pl.ds(k, L)]
        f = plsc.load_gather(idx_v, [sh])
        fused_v[pl.ds(k, L)] = f
        tgo_v[pl.ds(k, L)] = plsc.load_gather(tgt_v, [f])
        gro_v[pl.ds(k, L)] = plsc.load_gather(grp_v, [f])

    pltpu.sync_copy(tgo_v, tg_out.at[pl.ds(base, BPW)])
    pltpu.sync_copy(gro_v, gr_out.at[pl.ds(base, BPW)])

    @pl.loop(0, BPW, step=VCHUNK)
    def _(c):
        pltpu.sync_copy(video_hbm.at[fused_v.at[pl.ds(c, VCHUNK)]], vbuf)
        pltpu.sync_copy(vbuf, v_out.at[pl.ds(base + c, VCHUNK)])

    pltpu.sync_copy(audio_hbm.at[fused_v], abuf)
    pltpu.sync_copy(abuf, a_out.at[pl.ds(base, BPW)])

    @pl.loop(0, BPW, step=TCHUNK)
    def _(c):
        pltpu.sync_copy(text_hbm.at[fused_v.at[pl.ds(c, TCHUNK)]], tbuf)
        pltpu.sync_copy(tbuf, t_out.at[pl.ds(base + c, TCHUNK)])


@functools.cache
def _sc_gather_kernel():
    mesh = plsc.VectorSubcoreMesh(core_axis_name="c", subcore_axis_name="s")
    cp = pltpu.CompilerParams()
    if "needs_layout_passes" in pltpu.CompilerParams.__dataclass_fields__:
        cp = dataclasses.replace(cp, needs_layout_passes=False)
    return pl.kernel(
        _sc_gather_body,
        compiler_params=cp,
        out_type=[
        jax.ShapeDtypeStruct((BATCH, 16, 768), jnp.float32),
        jax.ShapeDtypeStruct((BATCH, AD), jnp.float32),
        jax.ShapeDtypeStruct((BATCH, TD), jnp.float32),
        jax.ShapeDtypeStruct((BATCH,), jnp.int32),
        jax.ShapeDtypeStruct((BATCH,), jnp.int32),
    ],
    mesh=mesh,
    scratch_types=[
        pltpu.VMEM((BATCH,), jnp.int32),
        pltpu.VMEM((BATCH,), jnp.int32),
        pltpu.VMEM((BATCH,), jnp.int32),
        pltpu.VMEM((BPW,), jnp.int32),
        pltpu.VMEM((BPW,), jnp.int32),
        pltpu.VMEM((BPW,), jnp.int32),
        pltpu.VMEM((BPW,), jnp.int32),
        pltpu.VMEM((VCHUNK, 16, 768), jnp.float32),
        pltpu.VMEM((BPW, AD), jnp.float32),
        pltpu.VMEM((TCHUNK, TD), jnp.float32),
    ],
)


def kernel(batch_video, batch_audio, batch_text, batch_target, batch_group):
    n_c = batch_group.shape[0]
    counts = jnp.bincount(batch_group, length=NUM_GROUP)
    weights = (counts.astype(jnp.float32) / n_c) ** TAU
    weights = weights / weights.sum()
    logw = jnp.log(weights)
    logit_row = logw[batch_group][None, :]

    key = jax.random.key(42)
    k_mult, k_perm = jax.random.split(key)
    gumbel_field = jax.random.gumbel(k_mult, (BATCH, BATCH), jnp.float32)
    shuffle_idx = jax.random.permutation(k_perm, BATCH).astype(jnp.int32)

    indices = _tc_argmax(gumbel_field, logit_row).reshape(BATCH)

    v, a, t, tg, gr = _sc_gather_kernel()(batch_video, batch_audio, batch_text,
                                          batch_target, batch_group, indices,
                                          shuffle_idx)
    return (v, a, t, tg, gr)


# speedup vs baseline: 2.7916x; 1.5232x over previous
import dataclasses
import functools

import jax
import jax.numpy as jnp
import numpy as np
from jax import lax
from jax.experimental import pallas as pl
from jax.experimental.pallas import tpu as pltpu
from jax.experimental.pallas import tpu_sc as plsc

BATCH = 1024
NUM_GROUP = 2
TAU = 0.2
VD = 16 * 768
AD = 128
TD = 768

NC = 2
NS = 16
L = 16
NW = NC * NS
BPW = BATCH // NW
VCHUNK = 8
TCHUNK = 16


def _argmax_body(g_ref, logit_ref, out_ref):
    v = g_ref[...] + logit_ref[...]
    m = jnp.max(v, axis=1, keepdims=True)
    cols = lax.broadcasted_iota(jnp.int32, v.shape, 1)
    masked = jnp.where(v == m, cols, BATCH)
    out_ref[...] = jnp.min(masked, axis=1, keepdims=True)


def _tc_argmax(gumbel_field, logit_row):
    return pl.pallas_call(
        _argmax_body,
        out_shape=jax.ShapeDtypeStruct((BATCH, 1), jnp.int32),
    )(gumbel_field, logit_row)


def _sc_gather_body(video_hbm, audio_hbm, text_hbm, tgt_hbm, grp_hbm, idx_hbm,
                    shuf_hbm, v_out, a_out, t_out, tg_out, gr_out,
                    idx_v, tgt_v, grp_v, shuf_v, fused_v, tgo_v, gro_v,
                    vbuf, abuf, tbuf):
    wid = lax.axis_index("s") * NC + lax.axis_index("c")
    base = wid * BPW

    pltpu.sync_copy(idx_hbm, idx_v)
    pltpu.sync_copy(tgt_hbm, tgt_v)
    pltpu.sync_copy(grp_hbm, grp_v)
    pltpu.sync_copy(shuf_hbm.at[pl.ds(base, BPW)], shuf_v)

    @pl.loop(0, BPW, step=L)
    def _(k):
        sh = shuf_v[pl.ds(k, L)]
        f = plsc.load_gather(idx_v, [sh])
        fused_v[pl.ds(k, L)] = f
        tgo_v[pl.ds(k, L)] = plsc.load_gather(tgt_v, [f])
        gro_v[pl.ds(k, L)] = plsc.load_gather(grp_v, [f])

    pltpu.sync_copy(tgo_v, tg_out.at[pl.ds(base, BPW)])
    pltpu.sync_copy(gro_v, gr_out.at[pl.ds(base, BPW)])

    @pl.loop(0, BPW, step=VCHUNK)
    def _(c):
        pltpu.sync_copy(video_hbm.at[fused_v.at[pl.ds(c, VCHUNK)]], vbuf)
        pltpu.sync_copy(vbuf, v_out.at[pl.ds(base + c, VCHUNK)])

    pltpu.sync_copy(audio_hbm.at[fused_v], abuf)
    pltpu.sync_copy(abuf, a_out.at[pl.ds(base, BPW)])

    @pl.loop(0, BPW, step=TCHUNK)
    def _(c):
        pltpu.sync_copy(text_hbm.at[fused_v.at[pl.ds(c, TCHUNK)]], tbuf)
        pltpu.sync_copy(tbuf, t_out.at[pl.ds(base + c, TCHUNK)])


@functools.cache
def _sc_gather_kernel():
    mesh = plsc.VectorSubcoreMesh(core_axis_name="c", subcore_axis_name="s")
    cp = pltpu.CompilerParams()
    if "needs_layout_passes" in pltpu.CompilerParams.__dataclass_fields__:
        cp = dataclasses.replace(cp, needs_layout_passes=False)
    return pl.kernel(
        _sc_gather_body,
        compiler_params=cp,
        out_type=[
        jax.ShapeDtypeStruct((BATCH, 16, 768), jnp.float32),
        jax.ShapeDtypeStruct((BATCH, AD), jnp.float32),
        jax.ShapeDtypeStruct((BATCH, TD), jnp.float32),
        jax.ShapeDtypeStruct((BATCH,), jnp.int32),
        jax.ShapeDtypeStruct((BATCH,), jnp.int32),
    ],
    mesh=mesh,
    scratch_types=[
        pltpu.VMEM((BATCH,), jnp.int32),
        pltpu.VMEM((BATCH,), jnp.int32),
        pltpu.VMEM((BATCH,), jnp.int32),
        pltpu.VMEM((BPW,), jnp.int32),
        pltpu.VMEM((BPW,), jnp.int32),
        pltpu.VMEM((BPW,), jnp.int32),
        pltpu.VMEM((BPW,), jnp.int32),
        pltpu.VMEM((VCHUNK, 16, 768), jnp.float32),
        pltpu.VMEM((BPW, AD), jnp.float32),
        pltpu.VMEM((TCHUNK, TD), jnp.float32),
    ],
)


@functools.cache
def _fixed_draws():
    with jax.ensure_compile_time_eval():
        key = jax.random.key(42)
        k_mult, k_perm = jax.random.split(key)
        g = np.asarray(jax.random.gumbel(k_mult, (BATCH, BATCH), jnp.float32))
        shuf = np.asarray(jax.random.permutation(k_perm, BATCH), dtype=np.int32)
    return g, shuf


def kernel(batch_video, batch_audio, batch_text, batch_target, batch_group):
    n_c = batch_group.shape[0]
    count1 = jnp.sum(batch_group)
    counts = jnp.stack([n_c - count1, count1])
    weights = (counts.astype(jnp.float32) / n_c) ** TAU
    weights = weights / weights.sum()
    logw = jnp.log(weights)
    logit_row = logw[batch_group][None, :]

    g_np, shuf_np = _fixed_draws()
    gumbel_field = jnp.asarray(g_np)
    shuffle_idx = jnp.asarray(shuf_np)

    indices = _tc_argmax(gumbel_field, logit_row).reshape(BATCH)

    v, a, t, tg, gr = _sc_gather_kernel()(batch_video, batch_audio, batch_text,
                                          batch_target, batch_group, indices,
                                          shuffle_idx)
    return (v, a, t, tg, gr)


# speedup vs baseline: 2.8411x; 1.0177x over previous
import dataclasses
import functools

import jax
import jax.numpy as jnp
import numpy as np
from jax import lax
from jax.experimental import pallas as pl
from jax.experimental.pallas import tpu as pltpu
from jax.experimental.pallas import tpu_sc as plsc

BATCH = 1024
NUM_GROUP = 2
TAU = 0.2
VD = 16 * 768
AD = 128
TD = 768

NC = 2
NS = 16
L = 16
NW = NC * NS
BPW = BATCH // NW
VCHUNK = 4
TCHUNK = 16


def _argmax_body(g_ref, logit_ref, out_ref):
    v = g_ref[...] + logit_ref[...]
    m = jnp.max(v, axis=1, keepdims=True)
    cols = lax.broadcasted_iota(jnp.int32, v.shape, 1)
    masked = jnp.where(v == m, cols, BATCH)
    out_ref[...] = jnp.min(masked, axis=1, keepdims=True)


def _tc_argmax(gumbel_field, logit_row):
    return pl.pallas_call(
        _argmax_body,
        out_shape=jax.ShapeDtypeStruct((BATCH, 1), jnp.int32),
    )(gumbel_field, logit_row)


def _sc_gather_body(video_hbm, audio_hbm, text_hbm, tgt_hbm, grp_hbm, idx_hbm,
                    shuf_hbm, v_out, a_out, t_out, tg_out, gr_out,
                    idx_v, tgt_v, grp_v, shuf_v, fused_v, shift_v, tgo_v, gro_v,
                    vbuf0, vbuf1, abuf, tbuf,
                    gsem0, gsem1, wsem0, wsem1):
    wid = lax.axis_index("s") * NC + lax.axis_index("c")
    base = wid * BPW

    pltpu.sync_copy(idx_hbm, idx_v)
    pltpu.sync_copy(tgt_hbm, tgt_v)
    pltpu.sync_copy(grp_hbm, grp_v)
    pltpu.sync_copy(shuf_hbm.at[pl.ds(base, BPW)], shuf_v)

    lanes = lax.iota(jnp.int32, L)
    for k in range(0, BPW, L):
        sh = shuf_v[pl.ds(k, L)]
        f = plsc.load_gather(idx_v, [sh])
        fused_v[pl.ds(k, L)] = f
        tgo_v[pl.ds(k, L)] = plsc.load_gather(tgt_v, [f])
        gro_v[pl.ds(k, L)] = plsc.load_gather(grp_v, [f])
    for k in range(0, BPW, L):
        src = jnp.minimum(lanes + (k + VCHUNK), BPW - 1)
        shift_v[pl.ds(k, L)] = plsc.load_gather(fused_v, [src])

    nchunk = BPW // VCHUNK
    bufs = (vbuf0, vbuf1)
    gsems = (gsem0, gsem1)
    wsems = (wsem0, wsem1)

    def chunk_idx(c):
        if c % 2 == 0:
            return fused_v.at[pl.ds(c * VCHUNK, VCHUNK)]
        return shift_v.at[pl.ds((c - 1) * VCHUNK, VCHUNK)]

    def start_gather(c):
        return pltpu.async_copy(video_hbm.at[chunk_idx(c)], bufs[c % 2],
                                gsems[c % 2])

    def start_write(c):
        return pltpu.async_copy(bufs[c % 2],
                                v_out.at[pl.ds(base + c * VCHUNK, VCHUNK)],
                                wsems[c % 2])

    g_h = [None] * nchunk
    w_h = [None] * nchunk
    g_h[0] = start_gather(0)
    g_h[1] = start_gather(1)

    pltpu.sync_copy(tgo_v, tg_out.at[pl.ds(base, BPW)])
    pltpu.sync_copy(gro_v, gr_out.at[pl.ds(base, BPW)])
    pltpu.sync_copy(audio_hbm.at[fused_v], abuf)
    pltpu.sync_copy(abuf, a_out.at[pl.ds(base, BPW)])
    for c in range(0, BPW, TCHUNK):
        pltpu.sync_copy(text_hbm.at[fused_v.at[pl.ds(c, TCHUNK)]], tbuf)
        pltpu.sync_copy(tbuf, t_out.at[pl.ds(base + c, TCHUNK)])

    for c in range(nchunk):
        g_h[c].wait()
        w_h[c] = start_write(c)
        if c + 2 < nchunk:
            w_h[c].wait()
            g_h[c + 2] = start_gather(c + 2)
    w_h[nchunk - 2].wait()
    w_h[nchunk - 1].wait()


@functools.cache
def _sc_gather_kernel():
    mesh = plsc.VectorSubcoreMesh(core_axis_name="c", subcore_axis_name="s")
    cp = pltpu.CompilerParams()
    if "needs_layout_passes" in pltpu.CompilerParams.__dataclass_fields__:
        cp = dataclasses.replace(cp, needs_layout_passes=False)
    return pl.kernel(
        _sc_gather_body,
        compiler_params=cp,
        out_type=[
        jax.ShapeDtypeStruct((BATCH, 16, 768), jnp.float32),
        jax.ShapeDtypeStruct((BATCH, AD), jnp.float32),
        jax.ShapeDtypeStruct((BATCH, TD), jnp.float32),
        jax.ShapeDtypeStruct((BATCH,), jnp.int32),
        jax.ShapeDtypeStruct((BATCH,), jnp.int32),
    ],
    mesh=mesh,
    scratch_types=[
        pltpu.VMEM((BATCH,), jnp.int32),
        pltpu.VMEM((BATCH,), jnp.int32),
        pltpu.VMEM((BATCH,), jnp.int32),
        pltpu.VMEM((BPW,), jnp.int32),
        pltpu.VMEM((BPW,), jnp.int32),
        pltpu.VMEM((BPW,), jnp.int32),
        pltpu.VMEM((BPW,), jnp.int32),
        pltpu.VMEM((BPW,), jnp.int32),
        pltpu.VMEM((VCHUNK, 16, 768), jnp.float32),
        pltpu.VMEM((VCHUNK, 16, 768), jnp.float32),
        pltpu.VMEM((BPW, AD), jnp.float32),
        pltpu.VMEM((TCHUNK, TD), jnp.float32),
        pltpu.SemaphoreType.DMA,
        pltpu.SemaphoreType.DMA,
        pltpu.SemaphoreType.DMA,
        pltpu.SemaphoreType.DMA,
    ],
)


@functools.cache
def _fixed_draws():
    with jax.ensure_compile_time_eval():
        key = jax.random.key(42)
        k_mult, k_perm = jax.random.split(key)
        g = np.asarray(jax.random.gumbel(k_mult, (BATCH, BATCH), jnp.float32))
        shuf = np.asarray(jax.random.permutation(k_perm, BATCH), dtype=np.int32)
    return g, shuf


def kernel(batch_video, batch_audio, batch_text, batch_target, batch_group):
    n_c = batch_group.shape[0]
    count1 = jnp.sum(batch_group)
    counts = jnp.stack([n_c - count1, count1])
    weights = (counts.astype(jnp.float32) / n_c) ** TAU
    weights = weights / weights.sum()
    logw = jnp.log(weights)
    logit_row = logw[batch_group][None, :]

    g_np, shuf_np = _fixed_draws()
    gumbel_field = jnp.asarray(g_np)
    shuffle_idx = jnp.asarray(shuf_np)

    indices = _tc_argmax(gumbel_field, logit_row).reshape(BATCH)

    v, a, t, tg, gr = _sc_gather_kernel()(batch_video, batch_audio, batch_text,
                                          batch_target, batch_group, indices,
                                          shuffle_idx)
    return (v, a, t, tg, gr)
